# baseline (speedup 1.0000x reference)
import functools

import jax
import jax.numpy as jnp
from jax import lax
from jax.experimental import pallas as pl
from jax.experimental.pallas import tpu as pltpu
from jax.experimental.pallas import tpu_sc as plsc

B, T, C, Y = 16, 4096, 512, 512
AUDIO_SR = 22050
HOP_LENGTH = 256
EMG_SR = 1000.0

NC = 2
NS = 16
NW = NC * NS
NQ = 4
FRQ = T // NQ
NSLOT = 96
TAIL_SLOT = 94
LEFT_SLOT = 95
NPASS = B * NQ // NW
CH = 64
NCHW = FRQ // CH
LANES = 16


def _prep_body(dur_ref, cnt_ref, cid_ref, wout_ref):
    b = pl.program_id(0)
    scale = jnp.float32(HOP_LENGTH / float(AUDIO_SR))
    fT = jnp.float32(T)

    d = jnp.squeeze(dur_ref[...], 0).astype(jnp.float32)
    durs = jnp.maximum(jnp.round(d * scale * EMG_SR), 0.0)

    i0 = lax.broadcasted_iota(jnp.int32, (Y, Y), 0)
    i1 = lax.broadcasted_iota(jnp.int32, (Y, Y), 1)
    tri_lo = (i1 <= i0).astype(jnp.float32)
    cum = lax.dot_general(tri_lo, durs, (((1,), (0,)), ((), ())),
                          preferred_element_type=jnp.float32)

    total = jnp.sum(durs)
    d_last = jnp.sum(lax.slice(durs, (Y - 1, 0), (Y, 1)))
    cpl = jnp.sum(lax.slice(cum, (Y - 2, 0), (Y - 1, 1)))
    last_new = jnp.maximum(d_last + (fT - total), 0.0)
    ri = lax.broadcasted_iota(jnp.int32, (Y, 1), 0)
    durs = jnp.where(ri == Y - 1, last_new, durs)
    cum = jnp.where(ri == Y - 1, cpl + last_new, cum)
    prev = cum - durs
    cnt = jnp.clip(cum, 0.0, fT) - jnp.clip(prev, 0.0, fT)
    cnt_ref[...] = jnp.expand_dims(cnt, 0)
    tstart = jnp.clip(cpl, 0.0, fT)

    nes = ((cnt > 0.0) & (ri != Y - 1)).astype(jnp.float32)
    s_col = lax.dot_general(tri_lo, nes, (((1,), (0,)), ((), ())),
                            preferred_element_type=jnp.float32)
    sb = [jnp.sum(nes * (prev < float(q * FRQ))) for q in range(NQ)]
    q_y = lax.shift_right_logical(prev.astype(jnp.int32), 10)
    sb_y = jnp.where(q_y == 0, sb[0],
                     jnp.where(q_y == 1, sb[1],
                               jnp.where(q_y == 2, sb[2], sb[3])))
    rank_col = s_col - 1.0 - sb_y

    t_row = lax.broadcasted_iota(jnp.int32, (1, T), 1).astype(jnp.float32)
    seg_f = jnp.zeros((1, T), jnp.float32)
    s_f = jnp.zeros((1, T), jnp.float32)
    for yc in range(0, Y, 64):
        cchunk = lax.slice(cum, (yc, 0), (yc + 64, 1))
        seg_f = seg_f + jnp.sum((cchunk <= t_row).astype(jnp.float32),
                                axis=0, keepdims=True)
        pchunk = lax.slice(prev, (yc, 0), (yc + 64, 1))
        nchunk = lax.slice(nes, (yc, 0), (yc + 64, 1))
        s_f = s_f + jnp.sum((pchunk <= t_row).astype(jnp.float32) * nchunk,
                            axis=0, keepdims=True)

    def _at(row, j):
        return jnp.sum(lax.slice(row, (0, j), (1, j + 1)))

    seg_i = seg_f.astype(jnp.int32)
    w_row = lax.shift_right_logical(
        lax.broadcasted_iota(jnp.int32, (1, T), 1), 10)
    sbnd = [_at(seg_i, q * FRQ - 1) for q in range(1, NQ)]
    leftb = jnp.zeros((1, T), jnp.bool_)
    for q in range(1, NQ):
        leftb = leftb | ((w_row == q) & (seg_i == sbnd[q - 1]))
    sb_f = jnp.where(w_row == 0, sb[0],
                     jnp.where(w_row == 1, sb[1],
                               jnp.where(w_row == 2, sb[2], sb[3])))
    rank_f = (s_f - 1.0 - sb_f).astype(jnp.int32)
    tail_f = t_row >= tstart
    cid = jnp.where(tail_f, TAIL_SLOT,
                    jnp.where(leftb, LEFT_SLOT, rank_f))
    cid_ref[...] = jnp.expand_dims(cid, 0)

    jj = lax.broadcasted_iota(jnp.int32, (Y, NSLOT), 1).astype(jnp.float32)
    j32 = lax.broadcasted_iota(jnp.int32, (1, NSLOT), 1)
    rank_b = jnp.broadcast_to(rank_col, (Y, NSLOT))
    y_b = jnp.broadcast_to(ri.astype(jnp.float32), (Y, NSLOT))
    rows = []
    for q in range(NQ):
        own = jnp.broadcast_to(nes * (q_y == q), (Y, NSLOT))
        ind = ((rank_b == jj) & (own > 0.0)).astype(jnp.float32)
        val = jnp.sum(ind * y_b, axis=0, keepdims=True)
        hit = jnp.sum(ind, axis=0, keepdims=True)
        wq = jnp.where(hit > 0.0, val, -1.0).astype(jnp.int32)
        if q == 0:
            ylft = jnp.int32(-1)
        else:
            s_at = _at(seg_i, q * FRQ)
            s_pre = sbnd[q - 1]
            ylft = jnp.where((s_at == s_pre) & (s_at != Y - 1), s_at, -1)
        wq = jnp.where(j32 == TAIL_SLOT, Y - 1,
                       jnp.where(j32 == LEFT_SLOT, ylft, wq))
        rows.append(wq)
    wout_ref[...] = jnp.expand_dims(jnp.concatenate(rows, axis=0), 0)


_prep_call = pl.pallas_call(
    _prep_body,
    grid=(B,),
    in_specs=[pl.BlockSpec((1, Y, 1), lambda b: (b, 0, 0))],
    out_specs=[
        pl.BlockSpec((1, Y, 1), lambda b: (b, 0, 0)),
        pl.BlockSpec((1, 1, T), lambda b: (b, 0, 0)),
        pl.BlockSpec((1, NQ, NSLOT), lambda b: (b, 0, 0)),
    ],
    out_shape=[
        jax.ShapeDtypeStruct((B, Y, 1), jnp.float32),
        jax.ShapeDtypeStruct((B, 1, T), jnp.int32),
        jax.ShapeDtypeStruct((B, NQ, NSLOT), jnp.int32),
    ],
)


def _sc_body(feats_hbm, cid_hbm, out_hbm, bufA, bufB, acc, cidv, semA, semB):
    c = lax.axis_index("c")
    s = lax.axis_index("s")
    w = c * NS + s
    zero16 = jnp.zeros((LANES,), jnp.float32)
    iota16 = lax.iota(jnp.int32, LANES)
    bufs = (bufA, bufB)
    sems = (semA, semB)

    for p in range(NPASS):
        task = p * NW + w
        b = task // NQ
        q = lax.rem(task, NQ)
        t0 = q * FRQ

        pltpu.sync_copy(cid_hbm.at[b, 0, pl.ds(t0, FRQ)], cidv)

        def _z(i, carry):
            for cc in range(C // LANES):
                acc[i, pl.ds(cc * LANES, LANES)] = zero16
            return carry
        lax.fori_loop(0, NSLOT, _z, 0)

        def _accumulate(buf, base):
            def _g(g, carry):
                gbase = g * LANES
                cvec = cidv[pl.ds(base + gbase, LANES)]
                for rl in range(LANES):
                    rot = jnp.mod(iota16 + rl, LANES)
                    cid = jnp.take(cvec, rot)[0]
                    for cc in range(C // LANES):
                        x = buf[gbase + rl, pl.ds(cc * LANES, LANES)]
                        plsc.addupdate(
                            acc.at[cid, pl.ds(cc * LANES, LANES)], x)
                return carry
            lax.fori_loop(0, CH // LANES, _g, 0)

        pltpu.async_copy(feats_hbm.at[b, pl.ds(t0, CH)], bufA, semA)

        def _pair(k2, carry):
            pltpu.async_copy(
                feats_hbm.at[b, pl.ds(t0 + (2 * k2 + 1) * CH, CH)],
                bufB, semB)
            pltpu.make_async_copy(
                feats_hbm.at[b, pl.ds(t0, CH)], bufA, semA).wait()
            _accumulate(bufA, 2 * k2 * CH)

            @pl.when(k2 < NCHW // 2 - 1)
            def _():
                pltpu.async_copy(
                    feats_hbm.at[b, pl.ds(t0 + (2 * k2 + 2) * CH, CH)],
                    bufA, semA)
            pltpu.make_async_copy(
                feats_hbm.at[b, pl.ds(t0, CH)], bufB, semB).wait()
            _accumulate(bufB, (2 * k2 + 1) * CH)
            return carry
        lax.fori_loop(0, NCHW // 2, _pair, 0)

        pltpu.sync_copy(acc, out_hbm.at[b, q])


@functools.cache
def _sc_call():
    return functools.partial(
        pl.kernel,
        out_type=jax.ShapeDtypeStruct((B, NQ, NSLOT, C), jnp.float32),
        mesh=plsc.VectorSubcoreMesh(
            core_axis_name="c", subcore_axis_name="s",
            num_cores=NC, num_subcores=NS),
        scratch_types=[
            pltpu.VMEM((CH, C), jnp.float32),
            pltpu.VMEM((CH, C), jnp.float32),
            pltpu.VMEM((NSLOT, C), jnp.float32),
            pltpu.VMEM((FRQ,), jnp.int32),
            pltpu.SemaphoreType.DMA,
            pltpu.SemaphoreType.DMA,
        ],
    )(_sc_body)


def _post_body(sums_ref, wout_ref, cnt_ref, out_ref):
    parts = jnp.squeeze(sums_ref[...], 0)
    w_row = jnp.squeeze(wout_ref[...], 0)
    ri = lax.broadcasted_iota(jnp.int32, (Y, 1), 0)
    p_mat = (jnp.broadcast_to(w_row, (Y, NQ * NSLOT)) == ri).astype(
        jnp.float32)
    out2 = lax.dot_general(p_mat, parts, (((1,), (0,)), ((), ())),
                           precision=lax.Precision.HIGHEST,
                           preferred_element_type=jnp.float32)
    cnt = jnp.squeeze(cnt_ref[...], 0)
    inv = jnp.where(cnt > 0.0, 1.0 / jnp.maximum(cnt, 1.0), 0.0)
    out_ref[...] = jnp.expand_dims(out2 * inv, 0)


_post_call = pl.pallas_call(
    _post_body,
    grid=(B,),
    in_specs=[
        pl.BlockSpec((1, NQ * NSLOT, C), lambda b: (b, 0, 0)),
        pl.BlockSpec((1, 1, NQ * NSLOT), lambda b: (b, 0, 0)),
        pl.BlockSpec((1, Y, 1), lambda b: (b, 0, 0)),
    ],
    out_specs=pl.BlockSpec((1, Y, C), lambda b: (b, 0, 0)),
    out_shape=jax.ShapeDtypeStruct((B, Y, C), jnp.float32),
)


def kernel(emg_feats, durations):
    cnt, cid, wout = _prep_call(durations[:, :, None])
    sums = _sc_call()(emg_feats, cid)
    return _post_call(sums.reshape(B, NQ * NSLOT, C),
                      wout.reshape(B, 1, NQ * NSLOT), cnt)


# speedup vs baseline: 1.6059x; 1.6059x over previous
import functools

import jax
import jax.numpy as jnp
from jax import lax
from jax.experimental import pallas as pl
from jax.experimental.pallas import tpu as pltpu
from jax.experimental.pallas import tpu_sc as plsc

B, T, C, Y = 16, 4096, 512, 512
AUDIO_SR = 22050
HOP_LENGTH = 256
EMG_SR = 1000.0

NC = 2
NS = 16
NW = NC * NS
NQ = 4
FRQ = T // NQ
NSLOT = 96
TAIL_SLOT = 94
LEFT_SLOT = 95
NPASS = B * NQ // NW
CH = 64
NCHW = FRQ // CH
LANES = 16


def _prep_body(dur_ref, cnt_ref, cid_ref, wout_ref):
    b = pl.program_id(0)
    scale = jnp.float32(HOP_LENGTH / float(AUDIO_SR))
    fT = jnp.float32(T)

    d = jnp.squeeze(dur_ref[...], 0).astype(jnp.float32)
    durs = jnp.maximum(jnp.round(d * scale * EMG_SR), 0.0)

    i0 = lax.broadcasted_iota(jnp.int32, (Y, Y), 0)
    i1 = lax.broadcasted_iota(jnp.int32, (Y, Y), 1)
    tri_lo = (i1 <= i0).astype(jnp.float32)
    cum = lax.dot_general(tri_lo, durs, (((1,), (0,)), ((), ())),
                          preferred_element_type=jnp.float32)

    total = jnp.sum(durs)
    d_last = jnp.sum(lax.slice(durs, (Y - 1, 0), (Y, 1)))
    cpl = jnp.sum(lax.slice(cum, (Y - 2, 0), (Y - 1, 1)))
    last_new = jnp.maximum(d_last + (fT - total), 0.0)
    ri = lax.broadcasted_iota(jnp.int32, (Y, 1), 0)
    durs = jnp.where(ri == Y - 1, last_new, durs)
    cum = jnp.where(ri == Y - 1, cpl + last_new, cum)
    prev = cum - durs
    cnt = jnp.clip(cum, 0.0, fT) - jnp.clip(prev, 0.0, fT)
    cnt_ref[...] = jnp.expand_dims(cnt, 0)
    tstart = jnp.clip(cpl, 0.0, fT)

    nes = ((cnt > 0.0) & (ri != Y - 1)).astype(jnp.float32)
    s_col = lax.dot_general(tri_lo, nes, (((1,), (0,)), ((), ())),
                            preferred_element_type=jnp.float32)
    sb = [jnp.sum(nes * (prev < float(q * FRQ))) for q in range(NQ)]
    q_y = lax.shift_right_logical(prev.astype(jnp.int32), 10)
    sb_y = jnp.where(q_y == 0, sb[0],
                     jnp.where(q_y == 1, sb[1],
                               jnp.where(q_y == 2, sb[2], sb[3])))
    rank_col = s_col - 1.0 - sb_y

    t_row = lax.broadcasted_iota(jnp.int32, (1, T), 1).astype(jnp.float32)
    seg_f = jnp.zeros((1, T), jnp.float32)
    s_f = jnp.zeros((1, T), jnp.float32)
    for yc in range(0, Y, 64):
        cchunk = lax.slice(cum, (yc, 0), (yc + 64, 1))
        seg_f = seg_f + jnp.sum((cchunk <= t_row).astype(jnp.float32),
                                axis=0, keepdims=True)
        pchunk = lax.slice(prev, (yc, 0), (yc + 64, 1))
        nchunk = lax.slice(nes, (yc, 0), (yc + 64, 1))
        s_f = s_f + jnp.sum((pchunk <= t_row).astype(jnp.float32) * nchunk,
                            axis=0, keepdims=True)

    def _at(row, j):
        return jnp.sum(lax.slice(row, (0, j), (1, j + 1)))

    seg_i = seg_f.astype(jnp.int32)
    w_row = lax.shift_right_logical(
        lax.broadcasted_iota(jnp.int32, (1, T), 1), 10)
    sbnd = [_at(seg_i, q * FRQ - 1) for q in range(1, NQ)]
    leftb = jnp.zeros((1, T), jnp.bool_)
    for q in range(1, NQ):
        leftb = leftb | ((w_row == q) & (seg_i == sbnd[q - 1]))
    sb_f = jnp.where(w_row == 0, sb[0],
                     jnp.where(w_row == 1, sb[1],
                               jnp.where(w_row == 2, sb[2], sb[3])))
    rank_f = (s_f - 1.0 - sb_f).astype(jnp.int32)
    tail_f = t_row >= tstart
    cid = jnp.where(tail_f, TAIL_SLOT,
                    jnp.where(leftb, LEFT_SLOT, rank_f))
    cid_ref[...] = jnp.expand_dims(cid, 0)

    jj = lax.broadcasted_iota(jnp.int32, (Y, NSLOT), 1).astype(jnp.float32)
    j32 = lax.broadcasted_iota(jnp.int32, (1, NSLOT), 1)
    rank_b = jnp.broadcast_to(rank_col, (Y, NSLOT))
    y_b = jnp.broadcast_to(ri.astype(jnp.float32), (Y, NSLOT))
    rows = []
    for q in range(NQ):
        own = jnp.broadcast_to(nes * (q_y == q), (Y, NSLOT))
        ind = ((rank_b == jj) & (own > 0.0)).astype(jnp.float32)
        val = jnp.sum(ind * y_b, axis=0, keepdims=True)
        hit = jnp.sum(ind, axis=0, keepdims=True)
        wq = jnp.where(hit > 0.0, val, -1.0).astype(jnp.int32)
        if q == 0:
            ylft = jnp.int32(-1)
        else:
            s_at = _at(seg_i, q * FRQ)
            s_pre = sbnd[q - 1]
            ylft = jnp.where((s_at == s_pre) & (s_at != Y - 1), s_at, -1)
        wq = jnp.where(j32 == TAIL_SLOT, Y - 1,
                       jnp.where(j32 == LEFT_SLOT, ylft, wq))
        rows.append(wq)
    wout_ref[...] = jnp.expand_dims(jnp.concatenate(rows, axis=0), 0)


_prep_call = pl.pallas_call(
    _prep_body,
    grid=(B,),
    in_specs=[pl.BlockSpec((1, Y, 1), lambda b: (b, 0, 0))],
    out_specs=[
        pl.BlockSpec((1, Y, 1), lambda b: (b, 0, 0)),
        pl.BlockSpec((1, 1, T), lambda b: (b, 0, 0)),
        pl.BlockSpec((1, NQ, NSLOT), lambda b: (b, 0, 0)),
    ],
    out_shape=[
        jax.ShapeDtypeStruct((B, Y, 1), jnp.float32),
        jax.ShapeDtypeStruct((B, 1, T), jnp.int32),
        jax.ShapeDtypeStruct((B, NQ, NSLOT), jnp.int32),
    ],
)


def _sc_body(feats_hbm, cid_hbm, out_hbm, bufA, bufB, acc, cidv, semA, semB):
    c = lax.axis_index("c")
    s = lax.axis_index("s")
    w = c * NS + s
    zero16 = jnp.zeros((LANES,), jnp.float32)
    iota16 = lax.iota(jnp.int32, LANES)
    bufs = (bufA, bufB)
    sems = (semA, semB)

    for p in range(NPASS):
        task = p * NW + w
        b = task // NQ
        q = lax.rem(task, NQ)
        t0 = q * FRQ

        pltpu.sync_copy(cid_hbm.at[b, 0, pl.ds(t0, FRQ)], cidv)

        def _z(i, carry):
            for cc in range(C // LANES):
                acc[i, pl.ds(cc * LANES, LANES)] = zero16
            return carry
        lax.fori_loop(0, NSLOT, _z, 0)

        def _accumulate(buf, base):
            def _r(r2, carry):
                r = r2 * 2
                rr = base + r
                g16 = (rr // LANES) * LANES
                cvec = cidv[pl.ds(g16, LANES)]
                for rl in range(2):
                    rot = jnp.mod(iota16 + (rr + rl - g16), LANES)
                    cid = jnp.take(cvec, rot)[0]
                    for cc in range(C // LANES):
                        x = buf[r + rl, pl.ds(cc * LANES, LANES)]
                        plsc.addupdate(
                            acc.at[cid, pl.ds(cc * LANES, LANES)], x)
                return carry
            lax.fori_loop(0, CH // 2, _r, 0)

        pltpu.async_copy(feats_hbm.at[b, pl.ds(t0, CH)], bufA, semA)

        def _pair(k2, carry):
            pltpu.async_copy(
                feats_hbm.at[b, pl.ds(t0 + (2 * k2 + 1) * CH, CH)],
                bufB, semB)
            pltpu.make_async_copy(
                feats_hbm.at[b, pl.ds(t0, CH)], bufA, semA).wait()
            _accumulate(bufA, 2 * k2 * CH)

            @pl.when(k2 < NCHW // 2 - 1)
            def _():
                pltpu.async_copy(
                    feats_hbm.at[b, pl.ds(t0 + (2 * k2 + 2) * CH, CH)],
                    bufA, semA)
            pltpu.make_async_copy(
                feats_hbm.at[b, pl.ds(t0, CH)], bufB, semB).wait()
            _accumulate(bufB, (2 * k2 + 1) * CH)
            return carry
        lax.fori_loop(0, NCHW // 2, _pair, 0)

        pltpu.sync_copy(acc, out_hbm.at[b, q])


@functools.cache
def _sc_call():
    return functools.partial(
        pl.kernel,
        out_type=jax.ShapeDtypeStruct((B, NQ, NSLOT, C), jnp.float32),
        mesh=plsc.VectorSubcoreMesh(
            core_axis_name="c", subcore_axis_name="s",
            num_cores=NC, num_subcores=NS),
        scratch_types=[
            pltpu.VMEM((CH, C), jnp.float32),
            pltpu.VMEM((CH, C), jnp.float32),
            pltpu.VMEM((NSLOT, C), jnp.float32),
            pltpu.VMEM((FRQ,), jnp.int32),
            pltpu.SemaphoreType.DMA,
            pltpu.SemaphoreType.DMA,
        ],
    )(_sc_body)


def _post_body(sums_ref, wout_ref, cnt_ref, out_ref):
    parts = jnp.squeeze(sums_ref[...], 0)
    w_row = jnp.squeeze(wout_ref[...], 0)
    ri = lax.broadcasted_iota(jnp.int32, (Y, 1), 0)
    p_mat = (jnp.broadcast_to(w_row, (Y, NQ * NSLOT)) == ri).astype(
        jnp.float32)
    out2 = lax.dot_general(p_mat, parts, (((1,), (0,)), ((), ())),
                           precision=lax.Precision.HIGHEST,
                           preferred_element_type=jnp.float32)
    cnt = jnp.squeeze(cnt_ref[...], 0)
    inv = jnp.where(cnt > 0.0, 1.0 / jnp.maximum(cnt, 1.0), 0.0)
    out_ref[...] = jnp.expand_dims(out2 * inv, 0)


_post_call = pl.pallas_call(
    _post_body,
    grid=(B,),
    in_specs=[
        pl.BlockSpec((1, NQ * NSLOT, C), lambda b: (b, 0, 0)),
        pl.BlockSpec((1, 1, NQ * NSLOT), lambda b: (b, 0, 0)),
        pl.BlockSpec((1, Y, 1), lambda b: (b, 0, 0)),
    ],
    out_specs=pl.BlockSpec((1, Y, C), lambda b: (b, 0, 0)),
    out_shape=jax.ShapeDtypeStruct((B, Y, C), jnp.float32),
)


def kernel(emg_feats, durations):
    cnt, cid, wout = _prep_call(durations[:, :, None])
    sums = _sc_call()(emg_feats, cid)
    return _post_call(sums.reshape(B, NQ * NSLOT, C),
                      wout.reshape(B, 1, NQ * NSLOT), cnt)


# speedup vs baseline: 1.6562x; 1.0313x over previous
import functools

import jax
import jax.numpy as jnp
from jax import lax
from jax.experimental import pallas as pl
from jax.experimental.pallas import tpu as pltpu
from jax.experimental.pallas import tpu_sc as plsc

B, T, C, Y = 16, 4096, 512, 512
AUDIO_SR = 22050
HOP_LENGTH = 256
EMG_SR = 1000.0

NC = 2
NS = 16
NW = NC * NS
NQ = 4
FRQ = T // NQ
NSLOT = 96
TAIL_SLOT = 94
LEFT_SLOT = 95
NPASS = B * NQ // NW
CH = 64
NCHW = FRQ // CH
LANES = 16


def _prep_body(dur_ref, cnt_ref, cid_ref, wout_ref):
    b = pl.program_id(0)
    scale = jnp.float32(HOP_LENGTH / float(AUDIO_SR))
    fT = jnp.float32(T)

    d = jnp.squeeze(dur_ref[...], 0).astype(jnp.float32)
    durs = jnp.maximum(jnp.round(d * scale * EMG_SR), 0.0)

    i0 = lax.broadcasted_iota(jnp.int32, (Y, Y), 0)
    i1 = lax.broadcasted_iota(jnp.int32, (Y, Y), 1)
    tri_lo = (i1 <= i0).astype(jnp.float32)
    cum = lax.dot_general(tri_lo, durs, (((1,), (0,)), ((), ())),
                          preferred_element_type=jnp.float32)

    total = jnp.sum(durs)
    d_last = jnp.sum(lax.slice(durs, (Y - 1, 0), (Y, 1)))
    cpl = jnp.sum(lax.slice(cum, (Y - 2, 0), (Y - 1, 1)))
    last_new = jnp.maximum(d_last + (fT - total), 0.0)
    ri = lax.broadcasted_iota(jnp.int32, (Y, 1), 0)
    durs = jnp.where(ri == Y - 1, last_new, durs)
    cum = jnp.where(ri == Y - 1, cpl + last_new, cum)
    prev = cum - durs
    cnt = jnp.clip(cum, 0.0, fT) - jnp.clip(prev, 0.0, fT)
    cnt_ref[...] = jnp.expand_dims(cnt, 0)
    tstart = jnp.clip(cpl, 0.0, fT)

    nes = ((cnt > 0.0) & (ri != Y - 1)).astype(jnp.float32)
    s_col = lax.dot_general(tri_lo, nes, (((1,), (0,)), ((), ())),
                            preferred_element_type=jnp.float32)
    sb = [jnp.sum(nes * (prev < float(q * FRQ))) for q in range(NQ)]
    q_y = lax.shift_right_logical(prev.astype(jnp.int32), 10)
    sb_y = jnp.where(q_y == 0, sb[0],
                     jnp.where(q_y == 1, sb[1],
                               jnp.where(q_y == 2, sb[2], sb[3])))
    rank_col = s_col - 1.0 - sb_y

    t_row = lax.broadcasted_iota(jnp.int32, (1, T), 1).astype(jnp.float32)
    seg_f = jnp.zeros((1, T), jnp.float32)
    s_f = jnp.zeros((1, T), jnp.float32)
    for yc in range(0, Y, 64):
        cchunk = lax.slice(cum, (yc, 0), (yc + 64, 1))
        seg_f = seg_f + jnp.sum((cchunk <= t_row).astype(jnp.float32),
                                axis=0, keepdims=True)
        pchunk = lax.slice(prev, (yc, 0), (yc + 64, 1))
        nchunk = lax.slice(nes, (yc, 0), (yc + 64, 1))
        s_f = s_f + jnp.sum((pchunk <= t_row).astype(jnp.float32) * nchunk,
                            axis=0, keepdims=True)

    def _at(row, j):
        return jnp.sum(lax.slice(row, (0, j), (1, j + 1)))

    seg_i = seg_f.astype(jnp.int32)
    w_row = lax.shift_right_logical(
        lax.broadcasted_iota(jnp.int32, (1, T), 1), 10)
    sbnd = [_at(seg_i, q * FRQ - 1) for q in range(1, NQ)]
    leftb = jnp.zeros((1, T), jnp.bool_)
    for q in range(1, NQ):
        leftb = leftb | ((w_row == q) & (seg_i == sbnd[q - 1]))
    sb_f = jnp.where(w_row == 0, sb[0],
                     jnp.where(w_row == 1, sb[1],
                               jnp.where(w_row == 2, sb[2], sb[3])))
    rank_f = (s_f - 1.0 - sb_f).astype(jnp.int32)
    tail_f = t_row >= tstart
    cid = jnp.where(tail_f, TAIL_SLOT,
                    jnp.where(leftb, LEFT_SLOT, rank_f))
    cid_ref[...] = jnp.expand_dims(cid, 0)

    jj = lax.broadcasted_iota(jnp.int32, (Y, NSLOT), 1).astype(jnp.float32)
    j32 = lax.broadcasted_iota(jnp.int32, (1, NSLOT), 1)
    rank_b = jnp.broadcast_to(rank_col, (Y, NSLOT))
    y_b = jnp.broadcast_to(ri.astype(jnp.float32), (Y, NSLOT))
    rows = []
    for q in range(NQ):
        own = jnp.broadcast_to(nes * (q_y == q), (Y, NSLOT))
        ind = ((rank_b == jj) & (own > 0.0)).astype(jnp.float32)
        val = jnp.sum(ind * y_b, axis=0, keepdims=True)
        hit = jnp.sum(ind, axis=0, keepdims=True)
        wq = jnp.where(hit > 0.0, val, -1.0).astype(jnp.int32)
        if q == 0:
            ylft = jnp.int32(-1)
        else:
            s_at = _at(seg_i, q * FRQ)
            s_pre = sbnd[q - 1]
            ylft = jnp.where((s_at == s_pre) & (s_at != Y - 1), s_at, -1)
        wq = jnp.where(j32 == TAIL_SLOT, Y - 1,
                       jnp.where(j32 == LEFT_SLOT, ylft, wq))
        rows.append(wq)
    wout_ref[...] = jnp.expand_dims(jnp.concatenate(rows, axis=0), 0)


_prep_call = pl.pallas_call(
    _prep_body,
    grid=(B,),
    in_specs=[pl.BlockSpec((1, Y, 1), lambda b: (b, 0, 0))],
    out_specs=[
        pl.BlockSpec((1, Y, 1), lambda b: (b, 0, 0)),
        pl.BlockSpec((1, 1, T), lambda b: (b, 0, 0)),
        pl.BlockSpec((1, NQ, NSLOT), lambda b: (b, 0, 0)),
    ],
    out_shape=[
        jax.ShapeDtypeStruct((B, Y, 1), jnp.float32),
        jax.ShapeDtypeStruct((B, 1, T), jnp.int32),
        jax.ShapeDtypeStruct((B, NQ, NSLOT), jnp.int32),
    ],
)


def _sc_body(feats_hbm, cid_hbm, out_hbm, bufA, bufB, acc, cidv, semA, semB):
    c = lax.axis_index("c")
    s = lax.axis_index("s")
    w = c * NS + s
    zero16 = jnp.zeros((LANES,), jnp.float32)
    iota16 = lax.iota(jnp.int32, LANES)
    bufs = (bufA, bufB)
    sems = (semA, semB)

    for p in range(NPASS):
        task = p * NW + w
        b = task // NQ
        q = lax.rem(task, NQ)
        t0 = q * FRQ

        pltpu.sync_copy(cid_hbm.at[b, 0, pl.ds(t0, FRQ)], cidv)

        def _z(i, carry):
            for cc in range(C // LANES):
                acc[i, pl.ds(cc * LANES, LANES)] = zero16
            return carry
        lax.fori_loop(0, NSLOT, _z, 0)

        def _accumulate(buf, base):
            def _r(r2, carry):
                r = r2 * 2
                rr = base + r
                g16 = (rr // LANES) * LANES
                cvec = cidv[pl.ds(g16, LANES)]
                for rl in range(2):
                    rot = jnp.mod(iota16 + (rr + rl - g16), LANES)
                    cid = jnp.take(cvec, rot)[0]
                    for cc in range(C // LANES):
                        x = buf[r + rl, pl.ds(cc * LANES, LANES)]
                        plsc.addupdate(
                            acc.at[cid, pl.ds(cc * LANES, LANES)], x)
                return carry
            lax.fori_loop(0, CH // 2, _r, 0)

        pltpu.async_copy(feats_hbm.at[b, pl.ds(t0, CH)], bufA, semA)

        def _pair(k2, carry):
            pltpu.async_copy(
                feats_hbm.at[b, pl.ds(t0 + (2 * k2 + 1) * CH, CH)],
                bufB, semB)
            pltpu.make_async_copy(
                feats_hbm.at[b, pl.ds(t0, CH)], bufA, semA).wait()
            _accumulate(bufA, 2 * k2 * CH)

            @pl.when(k2 < NCHW // 2 - 1)
            def _():
                pltpu.async_copy(
                    feats_hbm.at[b, pl.ds(t0 + (2 * k2 + 2) * CH, CH)],
                    bufA, semA)
            pltpu.make_async_copy(
                feats_hbm.at[b, pl.ds(t0, CH)], bufB, semB).wait()
            _accumulate(bufB, (2 * k2 + 1) * CH)
            return carry
        lax.fori_loop(0, NCHW // 2, _pair, 0)

        pltpu.sync_copy(acc, out_hbm.at[b, q])


@functools.cache
def _sc_call():
    return functools.partial(
        pl.kernel,
        out_type=jax.ShapeDtypeStruct((B, NQ, NSLOT, C), jnp.float32),
        mesh=plsc.VectorSubcoreMesh(
            core_axis_name="c", subcore_axis_name="s",
            num_cores=NC, num_subcores=NS),
        scratch_types=[
            pltpu.VMEM((CH, C), jnp.float32),
            pltpu.VMEM((CH, C), jnp.float32),
            pltpu.VMEM((NSLOT, C), jnp.float32),
            pltpu.VMEM((FRQ,), jnp.int32),
            pltpu.SemaphoreType.DMA,
            pltpu.SemaphoreType.DMA,
        ],
    )(_sc_body)


def _post_body(sums_ref, wout_ref, cnt_ref, out_ref):
    parts = jnp.squeeze(sums_ref[...], 0)
    w_row = jnp.squeeze(wout_ref[...], 0)
    ri = lax.broadcasted_iota(jnp.int32, (Y, 1), 0)
    p_mat = (jnp.broadcast_to(w_row, (Y, NQ * NSLOT)) == ri).astype(
        jnp.float32)
    hi = (parts.astype(jnp.bfloat16)).astype(jnp.float32)
    lo = parts - hi
    dn = (((1,), (0,)), ((), ()))
    out2 = (lax.dot_general(p_mat, hi, dn,
                            preferred_element_type=jnp.float32)
            + lax.dot_general(p_mat, lo, dn,
                              preferred_element_type=jnp.float32))
    cnt = jnp.squeeze(cnt_ref[...], 0)
    inv = jnp.where(cnt > 0.0, 1.0 / jnp.maximum(cnt, 1.0), 0.0)
    out_ref[...] = jnp.expand_dims(out2 * inv, 0)


_post_call = pl.pallas_call(
    _post_body,
    grid=(B,),
    in_specs=[
        pl.BlockSpec((1, NQ * NSLOT, C), lambda b: (b, 0, 0)),
        pl.BlockSpec((1, 1, NQ * NSLOT), lambda b: (b, 0, 0)),
        pl.BlockSpec((1, Y, 1), lambda b: (b, 0, 0)),
    ],
    out_specs=pl.BlockSpec((1, Y, C), lambda b: (b, 0, 0)),
    out_shape=jax.ShapeDtypeStruct((B, Y, C), jnp.float32),
)


def kernel(emg_feats, durations):
    cnt, cid, wout = _prep_call(durations[:, :, None])
    sums = _sc_call()(emg_feats, cid)
    return _post_call(sums.reshape(B, NQ * NSLOT, C),
                      wout.reshape(B, 1, NQ * NSLOT), cnt)


# speedup vs baseline: 1.6898x; 1.0203x over previous
import functools

import jax
import jax.numpy as jnp
from jax import lax
from jax.experimental import pallas as pl
from jax.experimental.pallas import tpu as pltpu
from jax.experimental.pallas import tpu_sc as plsc

B, T, C, Y = 16, 4096, 512, 512
AUDIO_SR = 22050
HOP_LENGTH = 256
EMG_SR = 1000.0

NC = 2
NS = 16
NW = NC * NS
NQ = 4
FRQ = T // NQ
NSLOT = 96
TAIL_SLOT = 94
LEFT_SLOT = 95
NPASS = B * NQ // NW
CH = 64
NCHW = FRQ // CH
LANES = 16


def _prep_body(dur_ref, cnt_ref, cid_ref, wout_ref):
    b = pl.program_id(0)
    scale = jnp.float32(HOP_LENGTH / float(AUDIO_SR))
    fT = jnp.float32(T)

    d = jnp.squeeze(dur_ref[...], 0).astype(jnp.float32)
    durs = jnp.maximum(jnp.round(d * scale * EMG_SR), 0.0)

    i0 = lax.broadcasted_iota(jnp.int32, (Y, Y), 0)
    i1 = lax.broadcasted_iota(jnp.int32, (Y, Y), 1)
    tri_lo = (i1 <= i0).astype(jnp.float32)
    cum = lax.dot_general(tri_lo, durs, (((1,), (0,)), ((), ())),
                          preferred_element_type=jnp.float32)

    total = jnp.sum(durs)
    d_last = jnp.sum(lax.slice(durs, (Y - 1, 0), (Y, 1)))
    cpl = jnp.sum(lax.slice(cum, (Y - 2, 0), (Y - 1, 1)))
    last_new = jnp.maximum(d_last + (fT - total), 0.0)
    ri = lax.broadcasted_iota(jnp.int32, (Y, 1), 0)
    durs = jnp.where(ri == Y - 1, last_new, durs)
    cum = jnp.where(ri == Y - 1, cpl + last_new, cum)
    prev = cum - durs
    cnt = jnp.clip(cum, 0.0, fT) - jnp.clip(prev, 0.0, fT)
    cnt_ref[...] = jnp.expand_dims(cnt, 0)
    tstart = jnp.clip(cpl, 0.0, fT)

    nes = ((cnt > 0.0) & (ri != Y - 1)).astype(jnp.float32)
    s_col = lax.dot_general(tri_lo, nes, (((1,), (0,)), ((), ())),
                            preferred_element_type=jnp.float32)
    sb = [jnp.sum(nes * (prev < float(q * FRQ))) for q in range(NQ)]
    q_y = lax.shift_right_logical(prev.astype(jnp.int32), 10)
    sb_y = jnp.where(q_y == 0, sb[0],
                     jnp.where(q_y == 1, sb[1],
                               jnp.where(q_y == 2, sb[2], sb[3])))
    rank_col = s_col - 1.0 - sb_y

    t_row = lax.broadcasted_iota(jnp.int32, (1, T), 1).astype(jnp.float32)
    seg_f = jnp.zeros((1, T), jnp.float32)
    s_f = jnp.zeros((1, T), jnp.float32)
    for yc in range(0, Y, 64):
        cchunk = lax.slice(cum, (yc, 0), (yc + 64, 1))
        seg_f = seg_f + jnp.sum((cchunk <= t_row).astype(jnp.float32),
                                axis=0, keepdims=True)
        pchunk = lax.slice(prev, (yc, 0), (yc + 64, 1))
        nchunk = lax.slice(nes, (yc, 0), (yc + 64, 1))
        s_f = s_f + jnp.sum((pchunk <= t_row).astype(jnp.float32) * nchunk,
                            axis=0, keepdims=True)

    def _at(row, j):
        return jnp.sum(lax.slice(row, (0, j), (1, j + 1)))

    seg_i = seg_f.astype(jnp.int32)
    w_row = lax.shift_right_logical(
        lax.broadcasted_iota(jnp.int32, (1, T), 1), 10)
    sbnd = [_at(seg_i, q * FRQ - 1) for q in range(1, NQ)]
    leftb = jnp.zeros((1, T), jnp.bool_)
    for q in range(1, NQ):
        leftb = leftb | ((w_row == q) & (seg_i == sbnd[q - 1]))
    sb_f = jnp.where(w_row == 0, sb[0],
                     jnp.where(w_row == 1, sb[1],
                               jnp.where(w_row == 2, sb[2], sb[3])))
    rank_f = (s_f - 1.0 - sb_f).astype(jnp.int32)
    tail_f = t_row >= tstart
    cid = jnp.where(tail_f, TAIL_SLOT,
                    jnp.where(leftb, LEFT_SLOT, rank_f))
    cid_ref[...] = jnp.expand_dims(cid, 0)

    jj = lax.broadcasted_iota(jnp.int32, (Y, NSLOT), 1).astype(jnp.float32)
    j32 = lax.broadcasted_iota(jnp.int32, (1, NSLOT), 1)
    rank_b = jnp.broadcast_to(rank_col, (Y, NSLOT))
    y_b = jnp.broadcast_to(ri.astype(jnp.float32), (Y, NSLOT))
    rows = []
    for q in range(NQ):
        own = jnp.broadcast_to(nes * (q_y == q), (Y, NSLOT))
        ind = ((rank_b == jj) & (own > 0.0)).astype(jnp.float32)
        val = jnp.sum(ind * y_b, axis=0, keepdims=True)
        hit = jnp.sum(ind, axis=0, keepdims=True)
        wq = jnp.where(hit > 0.0, val, -1.0).astype(jnp.int32)
        if q == 0:
            ylft = jnp.int32(-1)
        else:
            s_at = _at(seg_i, q * FRQ)
            s_pre = sbnd[q - 1]
            ylft = jnp.where((s_at == s_pre) & (s_at != Y - 1), s_at, -1)
        wq = jnp.where(j32 == TAIL_SLOT, Y - 1,
                       jnp.where(j32 == LEFT_SLOT, ylft, wq))
        rows.append(wq)
    wout_ref[...] = jnp.expand_dims(jnp.concatenate(rows, axis=0), 0)


_prep_call = pl.pallas_call(
    _prep_body,
    grid=(B,),
    in_specs=[pl.BlockSpec((1, Y, 1), lambda b: (b, 0, 0))],
    out_specs=[
        pl.BlockSpec((1, Y, 1), lambda b: (b, 0, 0)),
        pl.BlockSpec((1, 1, T), lambda b: (b, 0, 0)),
        pl.BlockSpec((1, NQ, NSLOT), lambda b: (b, 0, 0)),
    ],
    out_shape=[
        jax.ShapeDtypeStruct((B, Y, 1), jnp.float32),
        jax.ShapeDtypeStruct((B, 1, T), jnp.int32),
        jax.ShapeDtypeStruct((B, NQ, NSLOT), jnp.int32),
    ],
)


def _sc_body(feats_hbm, cid_hbm, out_hbm, bufA, bufB, acc, cidv, semA, semB):
    c = lax.axis_index("c")
    s = lax.axis_index("s")
    w = c * NS + s
    zero16 = jnp.zeros((LANES,), jnp.float32)
    iota16 = lax.iota(jnp.int32, LANES)
    bufs = (bufA, bufB)
    sems = (semA, semB)

    for p in range(NPASS):
        task = p * NW + w
        b = task // NQ
        q = lax.rem(task, NQ)
        t0 = q * FRQ

        pltpu.sync_copy(cid_hbm.at[b, 0, pl.ds(t0, FRQ)], cidv)

        def _z(i, carry):
            for cc in range(C // LANES):
                acc[i, pl.ds(cc * LANES, LANES)] = zero16
            return carry
        lax.fori_loop(0, NSLOT, _z, 0)

        def _accumulate(buf, base):
            def _r(r2, carry):
                r = r2 * 4
                rr = base + r
                g16 = (rr // LANES) * LANES
                cvec = cidv[pl.ds(g16, LANES)]
                for rl in range(4):
                    rot = jnp.mod(iota16 + (rr + rl - g16), LANES)
                    cid = jnp.take(cvec, rot)[0]
                    for cc in range(C // LANES):
                        x = buf[r + rl, pl.ds(cc * LANES, LANES)]
                        plsc.addupdate(
                            acc.at[cid, pl.ds(cc * LANES, LANES)], x)
                return carry
            lax.fori_loop(0, CH // 4, _r, 0)

        pltpu.async_copy(feats_hbm.at[b, pl.ds(t0, CH)], bufA, semA)

        def _pair(k2, carry):
            pltpu.async_copy(
                feats_hbm.at[b, pl.ds(t0 + (2 * k2 + 1) * CH, CH)],
                bufB, semB)
            pltpu.make_async_copy(
                feats_hbm.at[b, pl.ds(t0, CH)], bufA, semA).wait()
            _accumulate(bufA, 2 * k2 * CH)

            @pl.when(k2 < NCHW // 2 - 1)
            def _():
                pltpu.async_copy(
                    feats_hbm.at[b, pl.ds(t0 + (2 * k2 + 2) * CH, CH)],
                    bufA, semA)
            pltpu.make_async_copy(
                feats_hbm.at[b, pl.ds(t0, CH)], bufB, semB).wait()
            _accumulate(bufB, (2 * k2 + 1) * CH)
            return carry
        lax.fori_loop(0, NCHW // 2, _pair, 0)

        pltpu.sync_copy(acc, out_hbm.at[b, q])


@functools.cache
def _sc_call():
    return functools.partial(
        pl.kernel,
        out_type=jax.ShapeDtypeStruct((B, NQ, NSLOT, C), jnp.float32),
        mesh=plsc.VectorSubcoreMesh(
            core_axis_name="c", subcore_axis_name="s",
            num_cores=NC, num_subcores=NS),
        scratch_types=[
            pltpu.VMEM((CH, C), jnp.float32),
            pltpu.VMEM((CH, C), jnp.float32),
            pltpu.VMEM((NSLOT, C), jnp.float32),
            pltpu.VMEM((FRQ,), jnp.int32),
            pltpu.SemaphoreType.DMA,
            pltpu.SemaphoreType.DMA,
        ],
    )(_sc_body)


def _post_body(sums_ref, wout_ref, cnt_ref, out_ref):
    parts = jnp.squeeze(sums_ref[...], 0)
    w_row = jnp.squeeze(wout_ref[...], 0)
    ri = lax.broadcasted_iota(jnp.int32, (Y, 1), 0)
    p_mat = (jnp.broadcast_to(w_row, (Y, NQ * NSLOT)) == ri).astype(
        jnp.float32)
    hi = (parts.astype(jnp.bfloat16)).astype(jnp.float32)
    lo = parts - hi
    dn = (((1,), (0,)), ((), ()))
    out2 = (lax.dot_general(p_mat, hi, dn,
                            preferred_element_type=jnp.float32)
            + lax.dot_general(p_mat, lo, dn,
                              preferred_element_type=jnp.float32))
    cnt = jnp.squeeze(cnt_ref[...], 0)
    inv = jnp.where(cnt > 0.0, 1.0 / jnp.maximum(cnt, 1.0), 0.0)
    out_ref[...] = jnp.expand_dims(out2 * inv, 0)


_post_call = pl.pallas_call(
    _post_body,
    grid=(B,),
    in_specs=[
        pl.BlockSpec((1, NQ * NSLOT, C), lambda b: (b, 0, 0)),
        pl.BlockSpec((1, 1, NQ * NSLOT), lambda b: (b, 0, 0)),
        pl.BlockSpec((1, Y, 1), lambda b: (b, 0, 0)),
    ],
    out_specs=pl.BlockSpec((1, Y, C), lambda b: (b, 0, 0)),
    out_shape=jax.ShapeDtypeStruct((B, Y, C), jnp.float32),
)


def kernel(emg_feats, durations):
    cnt, cid, wout = _prep_call(durations[:, :, None])
    sums = _sc_call()(emg_feats, cid)
    return _post_call(sums.reshape(B, NQ * NSLOT, C),
                      wout.reshape(B, 1, NQ * NSLOT), cnt)
